# baseline (device time: 17428 ns/iter reference)
import numpy as np
import jax
import jax.numpy as jnp
from jax import lax
from jax.experimental import pallas as pl
from jax.experimental.pallas import tpu as pltpu

N_DEV = 4
Dh = 64
bf = jnp.bfloat16


def kernel(x, Wq, Wk, Wv, Wo):
    B, Sq, D = x.shape
    HD = Wq.shape[1]
    Hl = HD // Dh
    NBLK = 2 * B
    R = Sq // 2

    def body(x_ref, wq_ref, wk_ref, wv_ref, wo_ref, out_ref,
             acc_ref, sa_ref, sb_ref, ra_ref, rb_ref, rc_ref,
             send_sems, recv_sems, d_send_sems, d_recv_sems, out_sems):
        my = lax.axis_index("i")
        p1 = my ^ 1
        p2 = 3 - my
        pd = my ^ 2

        barrier = pltpu.get_barrier_semaphore()
        for nbr in (p1, p2, pd):
            pl.semaphore_signal(barrier, inc=1, device_id=(nbr,),
                                device_id_type=pl.DeviceIdType.MESH)

        srow = lax.broadcasted_iota(jnp.int32, (Sq, HD), 0)
        d = lax.broadcasted_iota(jnp.int32, (Sq, HD), 1) % Dh
        pair = (d // 2).astype(jnp.float32)
        invf = jnp.exp(pair * jnp.float32(-2.0 * np.log(10000.0) / Dh))
        ang = srow.astype(jnp.float32) * invf
        cos = jnp.cos(ang)
        sin = jnp.sin(ang)
        rr = lax.broadcasted_iota(jnp.int32, (HD, HD), 0)
        cc = lax.broadcasted_iota(jnp.int32, (HD, HD), 1)
        same = (rr // Dh) == (cc // Dh)
        ri = rr % Dh
        ci = cc % Dh
        Rb = (jnp.where(same & (ri == ci + 1) & (ci % 2 == 0), -1.0, 0.0)
              + jnp.where(same & (ci == ri + 1) & (ri % 2 == 0), 1.0, 0.0)
              ).astype(bf)

        wqb = wq_ref[:].astype(bf)
        wkb = wk_ref[:].astype(bf)
        wvb = wv_ref[:].astype(bf)
        wob = wo_ref[:].astype(bf)

        def exchange(stage, blk, src_ref, dst_ref, partner):
            return pltpu.make_async_remote_copy(
                src_ref=src_ref.at[blk],
                dst_ref=dst_ref.at[blk],
                send_sem=send_sems.at[stage, blk],
                recv_sem=recv_sems.at[stage, blk],
                device_id=(partner,),
                device_id_type=pl.DeviceIdType.MESH,
            )

        def partners(blk):
            return (p1, p2) if blk % 2 == 0 else (p2, p1)

        rdA = []
        rd3 = {}
        for b in range(B):
            xb = x_ref[b].astype(bf)
            q = jnp.dot(xb, wqb, preferred_element_type=jnp.float32)
            k = jnp.dot(xb, wkb, preferred_element_type=jnp.float32)
            v = jnp.dot(xb, wvb, preferred_element_type=jnp.float32)
            qr = q * cos + jnp.dot(q.astype(bf), Rb,
                                   preferred_element_type=jnp.float32) * sin
            kr = k * cos + jnp.dot(k.astype(bf), Rb,
                                   preferred_element_type=jnp.float32) * sin
            cols = []
            for h in range(Hl):
                c0 = h * Dh
                qh = qr[:, c0:c0 + Dh].astype(bf)
                kh = kr[:, c0:c0 + Dh].astype(bf)
                vh = v[:, c0:c0 + Dh].astype(bf)
                s = lax.dot_general(
                    qh, kh, (((1,), (1,)), ((), ())),
                    preferred_element_type=jnp.float32) * 0.125
                e = jnp.exp(s)
                denom = jnp.sum(e, axis=-1, keepdims=True)
                cols.append(jnp.dot(e.astype(bf), vh,
                                    preferred_element_type=jnp.float32) / denom)
            ctxb = jnp.concatenate(cols, axis=1)
            pb = jnp.dot(ctxb.astype(bf), wob,
                         preferred_element_type=jnp.float32)
            for h in range(2):
                blk = 2 * b + h
                pblk = pb[h * R:(h + 1) * R, :]
                acc_ref[blk] = pblk
                sa_ref[blk] = pblk.astype(bf)
                if blk == 0:
                    pl.semaphore_wait(barrier, 3)
                if b == 0:
                    rd = exchange(0, blk, sa_ref, ra_ref, partners(blk)[0])
                    rd.start()
                    rdA.append(rd)
                else:
                    k3 = blk - 2
                    r1 = exchange(0, blk, sa_ref, ra_ref, p1)
                    r2 = exchange(1, blk, sa_ref, rb_ref, p2)
                    rdg = pltpu.make_async_remote_copy(
                        src_ref=sa_ref.at[blk],
                        dst_ref=rc_ref.at[k3],
                        send_sem=d_send_sems.at[k3],
                        recv_sem=d_recv_sems.at[k3],
                        device_id=(pd,),
                        device_id_type=pl.DeviceIdType.MESH,
                    )
                    r1.start()
                    r2.start()
                    rdg.start()
                    rd3[blk] = (r1, r2, rdg)

        rdB = []
        for blk in range(2):
            rdA[blk].wait_recv()
            sum2 = acc_ref[blk] + ra_ref[blk].astype(jnp.float32)
            acc_ref[blk] = sum2
            sb_ref[blk] = sum2.astype(bf)
            rd = exchange(1, blk, sb_ref, rb_ref, partners(blk)[1])
            rd.start()
            rdB.append(rd)

        outcps = []

        def flush(blk, fin):
            acc_ref[blk] = fin
            b, h = divmod(blk, 2)
            cp = pltpu.make_async_copy(
                acc_ref.at[blk],
                out_ref.at[b, pl.ds(h * R, R), :],
                out_sems.at[blk],
            )
            cp.start()
            outcps.append(cp)

        for blk in range(2):
            rdB[blk].wait_recv()
            flush(blk, acc_ref[blk] + rb_ref[blk].astype(jnp.float32))
        for blk in range(2, NBLK):
            r1, r2, rdg = rd3[blk]
            r1.wait_recv()
            r2.wait_recv()
            rdg.wait_recv()
            flush(blk, acc_ref[blk]
                  + ra_ref[blk].astype(jnp.float32)
                  + rb_ref[blk].astype(jnp.float32)
                  + rc_ref[blk - 2].astype(jnp.float32))

        for cp in outcps:
            cp.wait()
        for rd in rdA + rdB:
            rd.wait_send()
        for blk in (2, 3):
            for rd in rd3[blk]:
                rd.wait_send()

    return pl.pallas_call(
        body,
        out_shape=jax.ShapeDtypeStruct((B, Sq, D), jnp.float32),
        in_specs=[pl.BlockSpec(memory_space=pltpu.VMEM)] * 5,
        out_specs=pl.BlockSpec(memory_space=pl.ANY),
        scratch_shapes=[
            pltpu.VMEM((NBLK, R, D), jnp.float32),
            pltpu.VMEM((NBLK, R, D), bf),
            pltpu.VMEM((NBLK, R, D), bf),
            pltpu.VMEM((NBLK, R, D), bf),
            pltpu.VMEM((NBLK, R, D), bf),
            pltpu.VMEM((2, R, D), bf),
            pltpu.SemaphoreType.DMA((2, NBLK)),
            pltpu.SemaphoreType.DMA((2, NBLK)),
            pltpu.SemaphoreType.DMA((2,)),
            pltpu.SemaphoreType.DMA((2,)),
            pltpu.SemaphoreType.DMA((NBLK,)),
        ],
        compiler_params=pltpu.CompilerParams(collective_id=0),
    )(x, Wq, Wk, Wv, Wo)
